# baseline (speedup 1.0000x reference)
import functools

import jax
import jax.numpy as jnp
from jax import lax
from jax.experimental import pallas as pl
from jax.experimental.pallas import tpu as pltpu
from jax.experimental.pallas import tpu_sc as plsc

B = 16384
D = 64
NC = 2
NS = 16
NW = NC * NS
B_PER_W = B // NW
CH = 128
NCH = B_PER_W // CH

BLK = 2048


def _sc_gather(users, items, eu_mlp, ei_mlp, eu_mf, ei_mf):
    mesh = plsc.VectorSubcoreMesh(core_axis_name="c", subcore_axis_name="s")
    row_t = jax.ShapeDtypeStruct((B, D), jnp.float32)

    @functools.partial(
        pl.kernel,
        mesh=mesh,
        out_type=[row_t, row_t, row_t, row_t],
        compiler_params=pltpu.CompilerParams(use_tc_tiling_on_sc=False),
        scratch_types=[
            pltpu.VMEM((B_PER_W,), jnp.int32),
            pltpu.VMEM((B_PER_W,), jnp.int32),
            pltpu.VMEM((B_PER_W, D), jnp.float32),
            pltpu.VMEM((B_PER_W, D), jnp.float32),
            pltpu.SemaphoreType.DMA,
        ],
    )
    def gather_kernel(users_hbm, items_hbm, eu_mlp_hbm, ei_mlp_hbm,
                      eu_mf_hbm, ei_mf_hbm,
                      o_umlp, o_imlp, o_umf, o_imf,
                      uidx_v, iidx_v, rows_u, rows_i, sem):
        wid = lax.axis_index("s") * NC + lax.axis_index("c")
        base = wid * B_PER_W
        pltpu.sync_copy(users_hbm.at[pl.ds(base, B_PER_W)], uidx_v)
        pltpu.sync_copy(items_hbm.at[pl.ds(base, B_PER_W)], iidx_v)

        for table_u, table_i, out_u, out_i in (
            (eu_mlp_hbm, ei_mlp_hbm, o_umlp, o_imlp),
            (eu_mf_hbm, ei_mf_hbm, o_umf, o_imf),
        ):
            copies = []
            for c in range(NCH):
                sl = pl.ds(c * CH, CH)
                copies.append(pltpu.async_copy(
                    table_u.at[uidx_v.at[sl]], rows_u.at[sl], sem))
                copies.append(pltpu.async_copy(
                    table_i.at[iidx_v.at[sl]], rows_i.at[sl], sem))
            for cp in copies:
                cp.wait()
            pltpu.sync_copy(rows_u, out_u.at[pl.ds(base, B_PER_W)])
            pltpu.sync_copy(rows_i, out_i.at[pl.ds(base, B_PER_W)])

    return gather_kernel(users, items, eu_mlp, ei_mlp, eu_mf, ei_mf)


def _mlp_body(umlp_ref, imlp_ref, umf_ref, imf_ref,
              w0u_ref, w0i_ref, b0_ref, w1_ref, b1_ref, w2_ref, b2_ref,
              wah_ref, wam_ref, ba_ref, out_ref):
    f32 = jnp.float32
    h = jnp.dot(umlp_ref[...], w0u_ref[...], preferred_element_type=f32)
    h += jnp.dot(imlp_ref[...], w0i_ref[...], preferred_element_type=f32)
    h = jnp.maximum(h + b0_ref[...], 0.0)
    h = jnp.maximum(
        jnp.dot(h, w1_ref[...], preferred_element_type=f32) + b1_ref[...], 0.0)
    h = jnp.maximum(
        jnp.dot(h, w2_ref[...], preferred_element_type=f32) + b2_ref[...], 0.0)
    mf = umf_ref[...] * imf_ref[...]
    logit = (jnp.sum(h * wah_ref[...], axis=1, keepdims=True)
             + jnp.sum(mf * wam_ref[...], axis=1, keepdims=True)
             + ba_ref[0, 0])
    out_ref[...] = jax.nn.sigmoid(logit)


def _tc_mlp(umlp, imlp, umf, imf, W0, b0, W1, b1, W2, b2, Wa, ba):
    L1, L2, L3 = W0.shape[1], W1.shape[1], W2.shape[1]
    w0u = W0[:D]
    w0i = W0[D:]
    wah = Wa[:L3].reshape(1, L3)
    wam = Wa[L3:].reshape(1, D)
    full = lambda shape: pl.BlockSpec(shape, lambda i: (0, 0))
    return pl.pallas_call(
        _mlp_body,
        grid=(B // BLK,),
        in_specs=[
            pl.BlockSpec((BLK, D), lambda i: (i, 0)),
            pl.BlockSpec((BLK, D), lambda i: (i, 0)),
            pl.BlockSpec((BLK, D), lambda i: (i, 0)),
            pl.BlockSpec((BLK, D), lambda i: (i, 0)),
            full((D, L1)), full((D, L1)), full((1, L1)),
            full((L1, L2)), full((1, L2)),
            full((L2, L3)), full((1, L3)),
            full((1, L3)), full((1, D)), full((1, 1)),
        ],
        out_specs=pl.BlockSpec((BLK, 1), lambda i: (i, 0)),
        out_shape=jax.ShapeDtypeStruct((B, 1), jnp.float32),
    )(umlp, imlp, umf, imf, w0u, w0i, b0.reshape(1, L1),
      W1, b1.reshape(1, L2), W2, b2.reshape(1, L3),
      wah, wam, ba.reshape(1, 1))


def kernel(users, items, eu_mlp, ei_mlp, eu_mf, ei_mf,
           W0, b0, W1, b1, W2, b2, Wa, ba):
    users = users.astype(jnp.int32)
    items = items.astype(jnp.int32)
    umlp, imlp, umf, imf = _sc_gather(users, items, eu_mlp, ei_mlp, eu_mf, ei_mf)
    return _tc_mlp(umlp, imlp, umf, imf, W0, b0, W1, b1, W2, b2, Wa, ba)


# speedup vs baseline: 2.3135x; 2.3135x over previous
import functools

import jax
import jax.numpy as jnp
from jax import lax
from jax.experimental import pallas as pl
from jax.experimental.pallas import tpu as pltpu
from jax.experimental.pallas import tpu_sc as plsc

B = 16384
D = 64
NC = 2
NS = 16
NW = NC * NS
B_PER_W = B // NW
CH = 128
NCH = B_PER_W // CH

KB = 4096
BLK = 2048


def _relayout_body(umlp_ref, umf_ref, imlp_ref, imf_ref, u_ref, i_ref):
    u_ref[:, :D] = umlp_ref[...].T
    u_ref[:, D:] = umf_ref[...].T
    i_ref[:, :D] = imlp_ref[...].T
    i_ref[:, D:] = imf_ref[...].T


def _tc_relayout(eu_mlp, ei_mlp, eu_mf, ei_mf):
    n = eu_mlp.shape[0]
    grid = (n + KB - 1) // KB
    tab_spec = pl.BlockSpec((D, KB), lambda i: (0, i))
    out_spec = pl.BlockSpec((KB, 2 * D), lambda i: (i, 0))
    return pl.pallas_call(
        _relayout_body,
        grid=(grid,),
        in_specs=[tab_spec] * 4,
        out_specs=[out_spec, out_spec],
        out_shape=[jax.ShapeDtypeStruct((n, 2 * D), jnp.float32)] * 2,
    )(eu_mlp.T, eu_mf.T, ei_mlp.T, ei_mf.T)


def _sc_gather(users, items, tab_u, tab_i):
    mesh = plsc.VectorSubcoreMesh(core_axis_name="c", subcore_axis_name="s")
    out_t = jax.ShapeDtypeStruct((B, 2 * D), jnp.float32)

    @functools.partial(
        pl.kernel,
        mesh=mesh,
        out_type=[out_t, out_t],
        scratch_types=[
            pltpu.VMEM((B_PER_W,), jnp.int32),
            pltpu.VMEM((B_PER_W,), jnp.int32),
            pltpu.VMEM((B_PER_W, 2 * D), jnp.float32),
            pltpu.SemaphoreType.DMA,
        ],
    )
    def gather_kernel(users_hbm, items_hbm, tab_u_hbm, tab_i_hbm,
                      o_u, o_i, uidx_v, iidx_v, rows_v, sem):
        wid = lax.axis_index("s") * NC + lax.axis_index("c")
        base = wid * B_PER_W
        pltpu.sync_copy(users_hbm.at[pl.ds(base, B_PER_W)], uidx_v)
        pltpu.sync_copy(items_hbm.at[pl.ds(base, B_PER_W)], iidx_v)

        for tab, idx_v, out in ((tab_u_hbm, uidx_v, o_u),
                                (tab_i_hbm, iidx_v, o_i)):
            copies = []
            for c in range(NCH):
                sl = pl.ds(c * CH, CH)
                copies.append(pltpu.async_copy(
                    tab.at[idx_v.at[sl]], rows_v.at[sl], sem))
            for cp in copies:
                cp.wait()
            pltpu.sync_copy(rows_v, out.at[pl.ds(base, B_PER_W)])

    return gather_kernel(users, items, tab_u, tab_i)


def _mlp_body(gu_ref, gi_ref,
              w0u_ref, w0i_ref, b0_ref, w1_ref, b1_ref, w2_ref, b2_ref,
              wah_ref, wam_ref, ba_ref, out_ref):
    f32 = jnp.float32
    gu = gu_ref[...]
    gi = gi_ref[...]
    h = jnp.dot(gu[:, :D], w0u_ref[...], preferred_element_type=f32)
    h += jnp.dot(gi[:, :D], w0i_ref[...], preferred_element_type=f32)
    h = jnp.maximum(h + b0_ref[...], 0.0)
    h = jnp.maximum(
        jnp.dot(h, w1_ref[...], preferred_element_type=f32) + b1_ref[...], 0.0)
    h = jnp.maximum(
        jnp.dot(h, w2_ref[...], preferred_element_type=f32) + b2_ref[...], 0.0)
    mf = gu[:, D:] * gi[:, D:]
    logit = (jnp.sum(h * wah_ref[...], axis=1, keepdims=True)
             + jnp.sum(mf * wam_ref[...], axis=1, keepdims=True)
             + ba_ref[0, 0])
    out_ref[...] = jax.nn.sigmoid(logit)


def _tc_mlp(gu, gi, W0, b0, W1, b1, W2, b2, Wa, ba):
    L1, L2, L3 = W0.shape[1], W1.shape[1], W2.shape[1]
    w0u = W0[:D]
    w0i = W0[D:]
    wah = Wa[:L3].reshape(1, L3)
    wam = Wa[L3:].reshape(1, D)
    full = lambda shape: pl.BlockSpec(shape, lambda i: (0, 0))
    return pl.pallas_call(
        _mlp_body,
        grid=(B // BLK,),
        in_specs=[
            pl.BlockSpec((BLK, 2 * D), lambda i: (i, 0)),
            pl.BlockSpec((BLK, 2 * D), lambda i: (i, 0)),
            full((D, L1)), full((D, L1)), full((1, L1)),
            full((L1, L2)), full((1, L2)),
            full((L2, L3)), full((1, L3)),
            full((1, L3)), full((1, D)), full((1, 1)),
        ],
        out_specs=pl.BlockSpec((BLK, 1), lambda i: (i, 0)),
        out_shape=jax.ShapeDtypeStruct((B, 1), jnp.float32),
    )(gu, gi, w0u, w0i, b0.reshape(1, L1),
      W1, b1.reshape(1, L2), W2, b2.reshape(1, L3),
      wah, wam, ba.reshape(1, 1))


def kernel(users, items, eu_mlp, ei_mlp, eu_mf, ei_mf,
           W0, b0, W1, b1, W2, b2, Wa, ba):
    users = users.astype(jnp.int32)
    items = items.astype(jnp.int32)
    tab_u, tab_i = _tc_relayout(eu_mlp, ei_mlp, eu_mf, ei_mf)
    gu, gi = _sc_gather(users, items, tab_u, tab_i)
    return _tc_mlp(gu, gi, W0, b0, W1, b1, W2, b2, Wa, ba)


# speedup vs baseline: 3.0463x; 1.3167x over previous
import functools

import jax
import jax.numpy as jnp
from jax import lax
from jax.experimental import pallas as pl
from jax.experimental.pallas import tpu as pltpu
from jax.experimental.pallas import tpu_sc as plsc

B = 16384
D = 64
NC = 2
NS = 16
NW = NC * NS
B_PER_W = B // NW
CH = 128
NCH = B_PER_W // CH

KB = 4096
BLK = 2048


def _relayout_body(umlp_ref, umf_ref, imlp_ref, imf_ref, u_ref, i_ref):
    u_ref[...] = jnp.concatenate([umlp_ref[...], umf_ref[...]], axis=0).T
    i_ref[...] = jnp.concatenate([imlp_ref[...], imf_ref[...]], axis=0).T


def _tc_relayout(eu_mlp, ei_mlp, eu_mf, ei_mf):
    n = eu_mlp.shape[0]
    grid = (n + KB - 1) // KB
    tab_spec = pl.BlockSpec((D, KB), lambda i: (0, i))
    out_spec = pl.BlockSpec((KB, 2 * D), lambda i: (i, 0))
    return pl.pallas_call(
        _relayout_body,
        grid=(grid,),
        in_specs=[tab_spec] * 4,
        out_specs=[out_spec, out_spec],
        out_shape=[jax.ShapeDtypeStruct((n, 2 * D), jnp.float32)] * 2,
    )(eu_mlp.T, eu_mf.T, ei_mlp.T, ei_mf.T)


def _sc_gather(users, items, tab_u, tab_i):
    mesh = plsc.VectorSubcoreMesh(core_axis_name="c", subcore_axis_name="s")
    out_t = jax.ShapeDtypeStruct((B, 2 * D), jnp.float32)

    @functools.partial(
        pl.kernel,
        mesh=mesh,
        out_type=[out_t, out_t],
        scratch_types=[
            pltpu.VMEM((B_PER_W,), jnp.int32),
            pltpu.VMEM((B_PER_W,), jnp.int32),
            pltpu.VMEM((B_PER_W, 2 * D), jnp.float32),
            pltpu.SemaphoreType.DMA,
        ],
    )
    def gather_kernel(users_hbm, items_hbm, tab_u_hbm, tab_i_hbm,
                      o_u, o_i, uidx_v, iidx_v, rows_v, sem):
        wid = lax.axis_index("s") * NC + lax.axis_index("c")
        base = wid * B_PER_W
        pltpu.sync_copy(users_hbm.at[pl.ds(base, B_PER_W)], uidx_v)
        pltpu.sync_copy(items_hbm.at[pl.ds(base, B_PER_W)], iidx_v)

        for tab, idx_v, out in ((tab_u_hbm, uidx_v, o_u),
                                (tab_i_hbm, iidx_v, o_i)):
            copies = []
            for c in range(NCH):
                sl = pl.ds(c * CH, CH)
                copies.append(pltpu.async_copy(
                    tab.at[idx_v.at[sl]], rows_v.at[sl], sem))
            for cp in copies:
                cp.wait()
            pltpu.sync_copy(rows_v, out.at[pl.ds(base, B_PER_W)])

    return gather_kernel(users, items, tab_u, tab_i)


def _mlp_body(gu_ref, gi_ref,
              w0u_ref, w0i_ref, b0_ref, w1_ref, b1_ref, w2_ref, b2_ref,
              wah_ref, wam_ref, ba_ref, out_ref):
    f32 = jnp.float32
    gu = gu_ref[...]
    gi = gi_ref[...]
    h = jnp.dot(gu[:, :D], w0u_ref[...], preferred_element_type=f32)
    h += jnp.dot(gi[:, :D], w0i_ref[...], preferred_element_type=f32)
    h = jnp.maximum(h + b0_ref[...], 0.0)
    h = jnp.maximum(
        jnp.dot(h, w1_ref[...], preferred_element_type=f32) + b1_ref[...], 0.0)
    h = jnp.maximum(
        jnp.dot(h, w2_ref[...], preferred_element_type=f32) + b2_ref[...], 0.0)
    mf = gu[:, D:] * gi[:, D:]
    logit = (jnp.sum(h * wah_ref[...], axis=1, keepdims=True)
             + jnp.sum(mf * wam_ref[...], axis=1, keepdims=True)
             + ba_ref[0, 0])
    out_ref[...] = jax.nn.sigmoid(logit)


def _tc_mlp(gu, gi, W0, b0, W1, b1, W2, b2, Wa, ba):
    L1, L2, L3 = W0.shape[1], W1.shape[1], W2.shape[1]
    w0u = W0[:D]
    w0i = W0[D:]
    wah = Wa[:L3].reshape(1, L3)
    wam = Wa[L3:].reshape(1, D)
    full = lambda shape: pl.BlockSpec(shape, lambda i: (0, 0))
    return pl.pallas_call(
        _mlp_body,
        grid=(B // BLK,),
        in_specs=[
            pl.BlockSpec((BLK, 2 * D), lambda i: (i, 0)),
            pl.BlockSpec((BLK, 2 * D), lambda i: (i, 0)),
            full((D, L1)), full((D, L1)), full((1, L1)),
            full((L1, L2)), full((1, L2)),
            full((L2, L3)), full((1, L3)),
            full((1, L3)), full((1, D)), full((1, 1)),
        ],
        out_specs=pl.BlockSpec((BLK, 1), lambda i: (i, 0)),
        out_shape=jax.ShapeDtypeStruct((B, 1), jnp.float32),
    )(gu, gi, w0u, w0i, b0.reshape(1, L1),
      W1, b1.reshape(1, L2), W2, b2.reshape(1, L3),
      wah, wam, ba.reshape(1, 1))


def kernel(users, items, eu_mlp, ei_mlp, eu_mf, ei_mf,
           W0, b0, W1, b1, W2, b2, Wa, ba):
    users = users.astype(jnp.int32)
    items = items.astype(jnp.int32)
    tab_u, tab_i = _tc_relayout(eu_mlp, ei_mlp, eu_mf, ei_mf)
    gu, gi = _sc_gather(users, items, tab_u, tab_i)
    return _tc_mlp(gu, gi, W0, b0, W1, b1, W2, b2, Wa, ba)


# speedup vs baseline: 3.8906x; 1.2772x over previous
import functools

import jax
import jax.numpy as jnp
from jax import lax
from jax.experimental import pallas as pl
from jax.experimental.pallas import tpu as pltpu
from jax.experimental.pallas import tpu_sc as plsc

B = 16384
D = 64
NC = 2
NS = 16
NW = NC * NS
B_PER_W = B // NW
CH = 128
NCH = B_PER_W // CH

KB = 4096
BLK = 2048

import numpy as np

_HI = np.uint32(0xFFFF0000)
_RND = np.uint32(0x7FFF)
_ONE = np.uint32(1)


def _pack_bf16_pair(hi_f32, lo_f32):
    u = lax.bitcast_convert_type(hi_f32, jnp.uint32)
    u = u + _RND + ((u >> 16) & _ONE)
    v = lax.bitcast_convert_type(lo_f32, jnp.uint32)
    v = v + _RND + ((v >> 16) & _ONE)
    return (u & _HI) | (v >> 16)


def _relayout_body(umlp_ref, umf_ref, imlp_ref, imf_ref, out_ref):
    pu = _pack_bf16_pair(umlp_ref[...], umf_ref[...])
    pi = _pack_bf16_pair(imlp_ref[...], imf_ref[...])
    packed = jnp.concatenate([pu, pi], axis=0)
    out_ref[...] = lax.bitcast_convert_type(packed, jnp.float32).T


def _tc_relayout(eu_mlp, ei_mlp, eu_mf, ei_mf):
    n = eu_mlp.shape[0]
    grid = (n + KB - 1) // KB
    tab_spec = pl.BlockSpec((D, KB), lambda i: (0, i))
    return pl.pallas_call(
        _relayout_body,
        grid=(grid,),
        in_specs=[tab_spec] * 4,
        out_specs=pl.BlockSpec((KB, 2 * D), lambda i: (i, 0)),
        out_shape=jax.ShapeDtypeStruct((n, 2 * D), jnp.float32),
    )(eu_mlp.T, eu_mf.T, ei_mlp.T, ei_mf.T)


def _sc_gather(users, items, tab):
    mesh = plsc.VectorSubcoreMesh(core_axis_name="c", subcore_axis_name="s")
    out_t = jax.ShapeDtypeStruct((B, 2 * D), jnp.float32)

    @functools.partial(
        pl.kernel,
        mesh=mesh,
        out_type=[out_t, out_t],
        scratch_types=[
            pltpu.VMEM((B_PER_W,), jnp.int32),
            pltpu.VMEM((B_PER_W,), jnp.int32),
            pltpu.VMEM((B_PER_W, 2 * D), jnp.float32),
            pltpu.SemaphoreType.DMA,
        ],
    )
    def gather_kernel(users_hbm, items_hbm, tab_hbm,
                      o_u, o_i, uidx_v, iidx_v, rows_v, sem):
        wid = lax.axis_index("s") * NC + lax.axis_index("c")
        base = wid * B_PER_W
        pltpu.sync_copy(users_hbm.at[pl.ds(base, B_PER_W)], uidx_v)
        pltpu.sync_copy(items_hbm.at[pl.ds(base, B_PER_W)], iidx_v)

        for idx_v, out in ((uidx_v, o_u), (iidx_v, o_i)):
            copies = []
            for c in range(NCH):
                sl = pl.ds(c * CH, CH)
                copies.append(pltpu.async_copy(
                    tab_hbm.at[idx_v.at[sl]], rows_v.at[sl], sem))
            for cp in copies:
                cp.wait()
            pltpu.sync_copy(rows_v, out.at[pl.ds(base, B_PER_W)])

    return gather_kernel(users, items, tab)


def _unpack(words_u32):
    hi = lax.bitcast_convert_type(words_u32 & _HI, jnp.float32)
    lo = lax.bitcast_convert_type(words_u32 << 16, jnp.float32)
    return hi, lo


def _mlp_body(gu_ref, gi_ref,
              w0u_ref, w0i_ref, b0_ref, w1_ref, b1_ref, w2_ref, b2_ref,
              wah_ref, wam_ref, ba_ref, out_ref):
    f32 = jnp.float32
    wu = lax.bitcast_convert_type(gu_ref[...][:, :D], jnp.uint32)
    wi = lax.bitcast_convert_type(gi_ref[...][:, D:], jnp.uint32)
    u_mlp, u_mf = _unpack(wu)
    i_mlp, i_mf = _unpack(wi)
    h = jnp.dot(u_mlp, w0u_ref[...], preferred_element_type=f32)
    h += jnp.dot(i_mlp, w0i_ref[...], preferred_element_type=f32)
    h = jnp.maximum(h + b0_ref[...], 0.0)
    h = jnp.maximum(
        jnp.dot(h, w1_ref[...], preferred_element_type=f32) + b1_ref[...], 0.0)
    h = jnp.maximum(
        jnp.dot(h, w2_ref[...], preferred_element_type=f32) + b2_ref[...], 0.0)
    mf = u_mf * i_mf
    logit = (jnp.sum(h * wah_ref[...], axis=1, keepdims=True)
             + jnp.sum(mf * wam_ref[...], axis=1, keepdims=True)
             + ba_ref[0, 0])
    out_ref[...] = jax.nn.sigmoid(logit)


def _tc_mlp(gu, gi, W0, b0, W1, b1, W2, b2, Wa, ba):
    L1, L2, L3 = W0.shape[1], W1.shape[1], W2.shape[1]
    w0u = W0[:D]
    w0i = W0[D:]
    wah = Wa[:L3].reshape(1, L3)
    wam = Wa[L3:].reshape(1, D)
    full = lambda shape: pl.BlockSpec(shape, lambda i: (0, 0))
    return pl.pallas_call(
        _mlp_body,
        grid=(B // BLK,),
        in_specs=[
            pl.BlockSpec((BLK, 2 * D), lambda i: (i, 0)),
            pl.BlockSpec((BLK, 2 * D), lambda i: (i, 0)),
            full((D, L1)), full((D, L1)), full((1, L1)),
            full((L1, L2)), full((1, L2)),
            full((L2, L3)), full((1, L3)),
            full((1, L3)), full((1, D)), full((1, 1)),
        ],
        out_specs=pl.BlockSpec((BLK, 1), lambda i: (i, 0)),
        out_shape=jax.ShapeDtypeStruct((B, 1), jnp.float32),
    )(gu, gi, w0u, w0i, b0.reshape(1, L1),
      W1, b1.reshape(1, L2), W2, b2.reshape(1, L3),
      wah, wam, ba.reshape(1, 1))


def kernel(users, items, eu_mlp, ei_mlp, eu_mf, ei_mf,
           W0, b0, W1, b1, W2, b2, Wa, ba):
    users = users.astype(jnp.int32)
    items = items.astype(jnp.int32)
    tab = _tc_relayout(eu_mlp, ei_mlp, eu_mf, ei_mf)
    gu, gi = _sc_gather(users, items, tab)
    return _tc_mlp(gu, gi, W0, b0, W1, b1, W2, b2, Wa, ba)


# speedup vs baseline: 4.0583x; 1.0431x over previous
import functools

import jax
import jax.numpy as jnp
from jax import lax
from jax.experimental import pallas as pl
from jax.experimental.pallas import tpu as pltpu
from jax.experimental.pallas import tpu_sc as plsc

B = 16384
D = 64
NC = 2
NS = 16
NW = NC * NS
B_PER_W = B // NW
CH = 128
NCH = B_PER_W // CH

KB = 8192
BLK = 2048

import numpy as np

_HI = np.uint32(0xFFFF0000)
_RND = np.uint32(0x7FFF)
_ONE = np.uint32(1)


def _pack_bf16_pair(hi_f32, lo_f32):
    u = lax.bitcast_convert_type(hi_f32, jnp.uint32)
    u = u + _RND + ((u >> 16) & _ONE)
    v = lax.bitcast_convert_type(lo_f32, jnp.uint32)
    v = v + _RND + ((v >> 16) & _ONE)
    return (u & _HI) | (v >> 16)


def _relayout_body(umlp_ref, umf_ref, imlp_ref, imf_ref, out_ref):
    pu = _pack_bf16_pair(umlp_ref[...], umf_ref[...])
    pi = _pack_bf16_pair(imlp_ref[...], imf_ref[...])
    packed = jnp.concatenate([pu, pi], axis=0)
    out_ref[...] = lax.bitcast_convert_type(packed, jnp.float32).T


def _tc_relayout(eu_mlp, ei_mlp, eu_mf, ei_mf):
    n = eu_mlp.shape[0]
    grid = (n + KB - 1) // KB
    tab_spec = pl.BlockSpec((D, KB), lambda i: (0, i))
    return pl.pallas_call(
        _relayout_body,
        grid=(grid,),
        in_specs=[tab_spec] * 4,
        out_specs=pl.BlockSpec((KB, 2 * D), lambda i: (i, 0)),
        out_shape=jax.ShapeDtypeStruct((n, 2 * D), jnp.float32),
    )(eu_mlp.T, eu_mf.T, ei_mlp.T, ei_mf.T)


def _sc_gather(users, items, tab):
    mesh = plsc.VectorSubcoreMesh(core_axis_name="c", subcore_axis_name="s")
    out_t = jax.ShapeDtypeStruct((B, 2 * D), jnp.float32)

    @functools.partial(
        pl.kernel,
        mesh=mesh,
        out_type=[out_t, out_t],
        scratch_types=[
            pltpu.VMEM((B_PER_W,), jnp.int32),
            pltpu.VMEM((B_PER_W,), jnp.int32),
            pltpu.VMEM((B_PER_W, 2 * D), jnp.float32),
            pltpu.SemaphoreType.DMA,
        ],
    )
    def gather_kernel(users_hbm, items_hbm, tab_hbm,
                      o_u, o_i, uidx_v, iidx_v, rows_v, sem):
        wid = lax.axis_index("s") * NC + lax.axis_index("c")
        base = wid * B_PER_W
        pltpu.sync_copy(users_hbm.at[pl.ds(base, B_PER_W)], uidx_v)
        pltpu.sync_copy(items_hbm.at[pl.ds(base, B_PER_W)], iidx_v)

        for idx_v, out in ((uidx_v, o_u), (iidx_v, o_i)):
            copies = []
            for c in range(NCH):
                sl = pl.ds(c * CH, CH)
                copies.append(pltpu.async_copy(
                    tab_hbm.at[idx_v.at[sl]], rows_v.at[sl], sem))
            for cp in copies:
                cp.wait()
            pltpu.sync_copy(rows_v, out.at[pl.ds(base, B_PER_W)])

    return gather_kernel(users, items, tab)


def _unpack(words_u32):
    hi = lax.bitcast_convert_type(words_u32 & _HI, jnp.float32)
    lo = lax.bitcast_convert_type(words_u32 << 16, jnp.float32)
    return hi, lo


def _mlp_body(gu_ref, gi_ref,
              w0u_ref, w0i_ref, b0_ref, w1_ref, b1_ref, w2_ref, b2_ref,
              wah_ref, wam_ref, ba_ref, out_ref):
    f32 = jnp.float32
    wu = lax.bitcast_convert_type(gu_ref[...][:, :D], jnp.uint32)
    wi = lax.bitcast_convert_type(gi_ref[...][:, D:], jnp.uint32)
    u_mlp, u_mf = _unpack(wu)
    i_mlp, i_mf = _unpack(wi)
    h = jnp.dot(u_mlp, w0u_ref[...], preferred_element_type=f32)
    h += jnp.dot(i_mlp, w0i_ref[...], preferred_element_type=f32)
    h = jnp.maximum(h + b0_ref[...], 0.0)
    h = jnp.maximum(
        jnp.dot(h, w1_ref[...], preferred_element_type=f32) + b1_ref[...], 0.0)
    h = jnp.maximum(
        jnp.dot(h, w2_ref[...], preferred_element_type=f32) + b2_ref[...], 0.0)
    mf = u_mf * i_mf
    logit = (jnp.sum(h * wah_ref[...], axis=1, keepdims=True)
             + jnp.sum(mf * wam_ref[...], axis=1, keepdims=True)
             + ba_ref[0, 0])
    out_ref[...] = jax.nn.sigmoid(logit)


def _tc_mlp(gu, gi, W0, b0, W1, b1, W2, b2, Wa, ba):
    L1, L2, L3 = W0.shape[1], W1.shape[1], W2.shape[1]
    w0u = W0[:D]
    w0i = W0[D:]
    wah = Wa[:L3].reshape(1, L3)
    wam = Wa[L3:].reshape(1, D)
    full = lambda shape: pl.BlockSpec(shape, lambda i: (0, 0))
    return pl.pallas_call(
        _mlp_body,
        grid=(B // BLK,),
        in_specs=[
            pl.BlockSpec((BLK, 2 * D), lambda i: (i, 0)),
            pl.BlockSpec((BLK, 2 * D), lambda i: (i, 0)),
            full((D, L1)), full((D, L1)), full((1, L1)),
            full((L1, L2)), full((1, L2)),
            full((L2, L3)), full((1, L3)),
            full((1, L3)), full((1, D)), full((1, 1)),
        ],
        out_specs=pl.BlockSpec((BLK, 1), lambda i: (i, 0)),
        out_shape=jax.ShapeDtypeStruct((B, 1), jnp.float32),
    )(gu, gi, w0u, w0i, b0.reshape(1, L1),
      W1, b1.reshape(1, L2), W2, b2.reshape(1, L3),
      wah, wam, ba.reshape(1, 1))


def kernel(users, items, eu_mlp, ei_mlp, eu_mf, ei_mf,
           W0, b0, W1, b1, W2, b2, Wa, ba):
    users = users.astype(jnp.int32)
    items = items.astype(jnp.int32)
    tab = _tc_relayout(eu_mlp, ei_mlp, eu_mf, ei_mf)
    gu, gi = _sc_gather(users, items, tab)
    return _tc_mlp(gu, gi, W0, b0, W1, b1, W2, b2, Wa, ba)


# speedup vs baseline: 4.0658x; 1.0018x over previous
import functools

import jax
import jax.numpy as jnp
from jax import lax
from jax.experimental import pallas as pl
from jax.experimental.pallas import tpu as pltpu
from jax.experimental.pallas import tpu_sc as plsc

B = 16384
D = 64
NC = 2
NS = 16
NW = NC * NS
B_PER_W = B // NW
CH = 128
NCH = B_PER_W // CH

KB = 8192
BLK = 2048

import numpy as np

_HI = np.uint32(0xFFFF0000)
_RND = np.uint32(0x7FFF)
_ONE = np.uint32(1)


def _pack_bf16_pair(hi_f32, lo_f32):
    u = lax.bitcast_convert_type(hi_f32, jnp.uint32)
    u = u + _RND + ((u >> 16) & _ONE)
    v = lax.bitcast_convert_type(lo_f32, jnp.uint32)
    v = v + _RND + ((v >> 16) & _ONE)
    return (u & _HI) | (v >> 16)


def _relayout_body(umlp_ref, umf_ref, imlp_ref, imf_ref, out_ref):
    pu = _pack_bf16_pair(umlp_ref[...], umf_ref[...])
    pi = _pack_bf16_pair(imlp_ref[...], imf_ref[...])
    packed = jnp.concatenate([pu, pi], axis=0)
    out_ref[...] = lax.bitcast_convert_type(packed, jnp.float32).T


def _tc_relayout(eu_mlp, ei_mlp, eu_mf, ei_mf):
    n = eu_mlp.shape[0]
    grid = (n + KB - 1) // KB
    tab_spec = pl.BlockSpec((D, KB), lambda i: (0, i))
    return pl.pallas_call(
        _relayout_body,
        grid=(grid,),
        in_specs=[tab_spec] * 4,
        out_specs=pl.BlockSpec((KB, 2 * D), lambda i: (i, 0)),
        out_shape=jax.ShapeDtypeStruct((n, 2 * D), jnp.float32),
        compiler_params=pltpu.CompilerParams(
            dimension_semantics=("parallel",)),
    )(eu_mlp.T, eu_mf.T, ei_mlp.T, ei_mf.T)


def _sc_gather(users, items, tab):
    mesh = plsc.VectorSubcoreMesh(core_axis_name="c", subcore_axis_name="s")
    out_t = jax.ShapeDtypeStruct((B, 2 * D), jnp.float32)

    @functools.partial(
        pl.kernel,
        mesh=mesh,
        out_type=[out_t, out_t],
        scratch_types=[
            pltpu.VMEM((B_PER_W,), jnp.int32),
            pltpu.VMEM((B_PER_W,), jnp.int32),
            pltpu.VMEM((B_PER_W, 2 * D), jnp.float32),
            pltpu.SemaphoreType.DMA,
        ],
    )
    def gather_kernel(users_hbm, items_hbm, tab_hbm,
                      o_u, o_i, uidx_v, iidx_v, rows_v, sem):
        wid = lax.axis_index("s") * NC + lax.axis_index("c")
        base = wid * B_PER_W
        pltpu.sync_copy(users_hbm.at[pl.ds(base, B_PER_W)], uidx_v)
        pltpu.sync_copy(items_hbm.at[pl.ds(base, B_PER_W)], iidx_v)

        for idx_v, out in ((uidx_v, o_u), (iidx_v, o_i)):
            copies = []
            for c in range(NCH):
                sl = pl.ds(c * CH, CH)
                copies.append(pltpu.async_copy(
                    tab_hbm.at[idx_v.at[sl]], rows_v.at[sl], sem))
            for cp in copies:
                cp.wait()
            pltpu.sync_copy(rows_v, out.at[pl.ds(base, B_PER_W)])

    return gather_kernel(users, items, tab)


def _unpack(words_u32):
    hi = lax.bitcast_convert_type(words_u32 & _HI, jnp.float32)
    lo = lax.bitcast_convert_type(words_u32 << 16, jnp.float32)
    return hi, lo


def _mlp_body(gu_ref, gi_ref,
              w0u_ref, w0i_ref, b0_ref, w1_ref, b1_ref, w2_ref, b2_ref,
              wah_ref, wam_ref, ba_ref, out_ref):
    f32 = jnp.float32
    wu = lax.bitcast_convert_type(gu_ref[...][:, :D], jnp.uint32)
    wi = lax.bitcast_convert_type(gi_ref[...][:, D:], jnp.uint32)
    u_mlp, u_mf = _unpack(wu)
    i_mlp, i_mf = _unpack(wi)
    h = jnp.dot(u_mlp, w0u_ref[...], preferred_element_type=f32)
    h += jnp.dot(i_mlp, w0i_ref[...], preferred_element_type=f32)
    h = jnp.maximum(h + b0_ref[...], 0.0)
    h = jnp.maximum(
        jnp.dot(h, w1_ref[...], preferred_element_type=f32) + b1_ref[...], 0.0)
    h = jnp.maximum(
        jnp.dot(h, w2_ref[...], preferred_element_type=f32) + b2_ref[...], 0.0)
    mf = u_mf * i_mf
    logit = (jnp.sum(h * wah_ref[...], axis=1, keepdims=True)
             + jnp.sum(mf * wam_ref[...], axis=1, keepdims=True)
             + ba_ref[0, 0])
    out_ref[...] = jax.nn.sigmoid(logit)


def _tc_mlp(gu, gi, W0, b0, W1, b1, W2, b2, Wa, ba):
    L1, L2, L3 = W0.shape[1], W1.shape[1], W2.shape[1]
    w0u = W0[:D]
    w0i = W0[D:]
    wah = Wa[:L3].reshape(1, L3)
    wam = Wa[L3:].reshape(1, D)
    full = lambda shape: pl.BlockSpec(shape, lambda i: (0, 0))
    return pl.pallas_call(
        _mlp_body,
        grid=(B // BLK,),
        in_specs=[
            pl.BlockSpec((BLK, 2 * D), lambda i: (i, 0)),
            pl.BlockSpec((BLK, 2 * D), lambda i: (i, 0)),
            full((D, L1)), full((D, L1)), full((1, L1)),
            full((L1, L2)), full((1, L2)),
            full((L2, L3)), full((1, L3)),
            full((1, L3)), full((1, D)), full((1, 1)),
        ],
        out_specs=pl.BlockSpec((BLK, 1), lambda i: (i, 0)),
        out_shape=jax.ShapeDtypeStruct((B, 1), jnp.float32),
    )(gu, gi, w0u, w0i, b0.reshape(1, L1),
      W1, b1.reshape(1, L2), W2, b2.reshape(1, L3),
      wah, wam, ba.reshape(1, 1))


def kernel(users, items, eu_mlp, ei_mlp, eu_mf, ei_mf,
           W0, b0, W1, b1, W2, b2, Wa, ba):
    users = users.astype(jnp.int32)
    items = items.astype(jnp.int32)
    tab = _tc_relayout(eu_mlp, ei_mlp, eu_mf, ei_mf)
    gu, gi = _sc_gather(users, items, tab)
    return _tc_mlp(gu, gi, W0, b0, W1, b1, W2, b2, Wa, ba)


# speedup vs baseline: 4.1646x; 1.0243x over previous
import functools

import jax
import jax.numpy as jnp
from jax import lax
from jax.experimental import pallas as pl
from jax.experimental.pallas import tpu as pltpu
from jax.experimental.pallas import tpu_sc as plsc

B = 16384
D = 64
NC = 2
NS = 16
NW = NC * NS
B_PER_W = B // NW
CH = 128
NCH = B_PER_W // CH

KB = 16384
BLK = 4096

import numpy as np

_HI = np.uint32(0xFFFF0000)
_RND = np.uint32(0x7FFF)
_ONE = np.uint32(1)


def _pack_bf16_pair(hi_f32, lo_f32):
    u = lax.bitcast_convert_type(hi_f32, jnp.uint32)
    u = u + _RND + ((u >> 16) & _ONE)
    v = lax.bitcast_convert_type(lo_f32, jnp.uint32)
    v = v + _RND + ((v >> 16) & _ONE)
    return (u & _HI) | (v >> 16)


def _relayout_body(umlp_ref, umf_ref, imlp_ref, imf_ref, out_ref):
    pu = _pack_bf16_pair(umlp_ref[...], umf_ref[...])
    pi = _pack_bf16_pair(imlp_ref[...], imf_ref[...])
    packed = jnp.concatenate([pu, pi], axis=0)
    out_ref[...] = lax.bitcast_convert_type(packed, jnp.float32).T


def _tc_relayout(eu_mlp, ei_mlp, eu_mf, ei_mf):
    n = eu_mlp.shape[0]
    grid = (n + KB - 1) // KB
    tab_spec = pl.BlockSpec((D, KB), lambda i: (0, i))
    return pl.pallas_call(
        _relayout_body,
        grid=(grid,),
        in_specs=[tab_spec] * 4,
        out_specs=pl.BlockSpec((KB, 2 * D), lambda i: (i, 0)),
        out_shape=jax.ShapeDtypeStruct((n, 2 * D), jnp.float32),
        compiler_params=pltpu.CompilerParams(
            dimension_semantics=("parallel",)),
    )(eu_mlp.T, eu_mf.T, ei_mlp.T, ei_mf.T)


def _sc_gather(users, items, tab):
    mesh = plsc.VectorSubcoreMesh(core_axis_name="c", subcore_axis_name="s")
    out_t = jax.ShapeDtypeStruct((B, 2 * D), jnp.float32)

    @functools.partial(
        pl.kernel,
        mesh=mesh,
        out_type=[out_t, out_t],
        scratch_types=[
            pltpu.VMEM((B_PER_W,), jnp.int32),
            pltpu.VMEM((B_PER_W,), jnp.int32),
            pltpu.VMEM((B_PER_W, 2 * D), jnp.float32),
            pltpu.SemaphoreType.DMA,
        ],
    )
    def gather_kernel(users_hbm, items_hbm, tab_hbm,
                      o_u, o_i, uidx_v, iidx_v, rows_v, sem):
        wid = lax.axis_index("s") * NC + lax.axis_index("c")
        base = wid * B_PER_W
        pltpu.sync_copy(users_hbm.at[pl.ds(base, B_PER_W)], uidx_v)
        pltpu.sync_copy(items_hbm.at[pl.ds(base, B_PER_W)], iidx_v)

        for idx_v, out in ((uidx_v, o_u), (iidx_v, o_i)):
            copies = []
            for c in range(NCH):
                sl = pl.ds(c * CH, CH)
                copies.append(pltpu.async_copy(
                    tab_hbm.at[idx_v.at[sl]], rows_v.at[sl], sem))
            for cp in copies:
                cp.wait()
            pltpu.sync_copy(rows_v, out.at[pl.ds(base, B_PER_W)])

    return gather_kernel(users, items, tab)


def _unpack(words_u32):
    hi = lax.bitcast_convert_type(words_u32 & _HI, jnp.float32)
    lo = lax.bitcast_convert_type(words_u32 << 16, jnp.float32)
    return hi, lo


def _mlp_body(gu_ref, gi_ref,
              w0u_ref, w0i_ref, b0_ref, w1_ref, b1_ref, w2_ref, b2_ref,
              wah_ref, wam_ref, ba_ref, out_ref):
    f32 = jnp.float32
    wu = lax.bitcast_convert_type(gu_ref[...][:, :D], jnp.uint32)
    wi = lax.bitcast_convert_type(gi_ref[...][:, D:], jnp.uint32)
    u_mlp, u_mf = _unpack(wu)
    i_mlp, i_mf = _unpack(wi)
    h = jnp.dot(u_mlp, w0u_ref[...], preferred_element_type=f32)
    h += jnp.dot(i_mlp, w0i_ref[...], preferred_element_type=f32)
    h = jnp.maximum(h + b0_ref[...], 0.0)
    h = jnp.maximum(
        jnp.dot(h, w1_ref[...], preferred_element_type=f32) + b1_ref[...], 0.0)
    h = jnp.maximum(
        jnp.dot(h, w2_ref[...], preferred_element_type=f32) + b2_ref[...], 0.0)
    mf = u_mf * i_mf
    logit = (jnp.sum(h * wah_ref[...], axis=1, keepdims=True)
             + jnp.sum(mf * wam_ref[...], axis=1, keepdims=True)
             + ba_ref[0, 0])
    out_ref[...] = jax.nn.sigmoid(logit)


def _tc_mlp(gu, gi, W0, b0, W1, b1, W2, b2, Wa, ba):
    L1, L2, L3 = W0.shape[1], W1.shape[1], W2.shape[1]
    w0u = W0[:D]
    w0i = W0[D:]
    wah = Wa[:L3].reshape(1, L3)
    wam = Wa[L3:].reshape(1, D)
    full = lambda shape: pl.BlockSpec(shape, lambda i: (0, 0))
    return pl.pallas_call(
        _mlp_body,
        grid=(B // BLK,),
        in_specs=[
            pl.BlockSpec((BLK, 2 * D), lambda i: (i, 0)),
            pl.BlockSpec((BLK, 2 * D), lambda i: (i, 0)),
            full((D, L1)), full((D, L1)), full((1, L1)),
            full((L1, L2)), full((1, L2)),
            full((L2, L3)), full((1, L3)),
            full((1, L3)), full((1, D)), full((1, 1)),
        ],
        out_specs=pl.BlockSpec((BLK, 1), lambda i: (i, 0)),
        out_shape=jax.ShapeDtypeStruct((B, 1), jnp.float32),
    )(gu, gi, w0u, w0i, b0.reshape(1, L1),
      W1, b1.reshape(1, L2), W2, b2.reshape(1, L3),
      wah, wam, ba.reshape(1, 1))


def kernel(users, items, eu_mlp, ei_mlp, eu_mf, ei_mf,
           W0, b0, W1, b1, W2, b2, Wa, ba):
    users = users.astype(jnp.int32)
    items = items.astype(jnp.int32)
    tab = _tc_relayout(eu_mlp, ei_mlp, eu_mf, ei_mf)
    gu, gi = _sc_gather(users, items, tab)
    return _tc_mlp(gu, gi, W0, b0, W1, b1, W2, b2, Wa, ba)
